# baseline (speedup 1.0000x reference)
import jax
import jax.numpy as jnp
from jax import lax
from jax.experimental import pallas as pl
from jax.experimental.pallas import tpu as pltpu
from jax.experimental.pallas import tpu_sc as plsc

D = 1024
V = 100000
VH = V // 2
CGW = 12512
CG_OWN = 12496
NRG = 4
RPT = D // NRG
RG = 4
NG = RPT // RG
NS = RPT // 16
AW = 3136
NJ = CGW // 16
NJA = AW // 16


def _sc_body(x_hbm, w_hbm, b_hbm, vals_hbm, idx_hbm,
             xv, acc, wb0, wb1, rd, st_v, st_i, shacc_all,
             sem0, sem1):
    cid = lax.axis_index("c")
    sid = lax.axis_index("s")
    wid = cid * 16 + sid
    rgrp = lax.rem(sid, NRG)
    cgrp = sid // NRG
    rbase = rgrp * RPT
    half = cid * VH
    cg_lo = half + cgrp * CG_OWN

    pltpu.sync_copy(x_hbm, xv)

    @pl.when(rgrp == 0)
    def _():
        pltpu.sync_copy(b_hbm.at[pl.ds(cg_lo, CGW)], acc)

    @pl.when(rgrp != 0)
    def _():
        @plsc.parallel_loop(0, NJ, unroll=4)
        def _zero(j):
            acc[pl.ds(j * 16, 16)] = jnp.zeros((16,), jnp.float32)

    def start(g, buf, sem):
        pltpu.async_copy(
            w_hbm.at[pl.ds(rbase + g * RG, RG), pl.ds(cg_lo, CGW)],
            buf, sem)

    def wait_g(g, buf, sem):
        pltpu.make_async_copy(
            w_hbm.at[pl.ds(rbase + g * RG, RG), pl.ds(cg_lo, CGW)],
            buf, sem).wait()

    start(0, wb0, sem0)
    start(1, wb1, sem1)

    def sbody(s, _):
        xg = xv[pl.ds(rbase + s * 16, 16)]
        xs = [xg[i] for i in range(16)]
        for k in range(4):
            g = s * 4 + k
            buf, sem = (wb0, sem0) if k % 2 == 0 else (wb1, sem1)
            wait_g(g, buf, sem)
            x4 = xs[4 * k:4 * k + 4]

            @plsc.parallel_loop(0, NJ, unroll=4)
            def _fma(j):
                sl = pl.ds(j * 16, 16)
                p = x4[0] * buf[0, sl] + x4[1] * buf[1, sl]
                q = x4[2] * buf[2, sl] + x4[3] * buf[3, sl]
                plsc.addupdate(acc.at[sl], p + q)

            @pl.when(g + 2 < NG)
            def _():
                start(g + 2, buf, sem)
        return 0

    lax.fori_loop(0, NS, sbody, 0)

    pltpu.sync_copy(acc, shacc_all.at[sid])
    plsc.subcore_barrier()

    @pl.when(rgrp == 0)
    def _():
        for r in range(1, NRG):
            pltpu.sync_copy(shacc_all.at[cgrp * NRG + r], wb0.at[r - 1])

        @plsc.parallel_loop(0, NJ, unroll=4)
        def _red(j):
            sl = pl.ds(j * 16, 16)
            plsc.addupdate(acc.at[sl], (wb0[0, sl] + wb0[1, sl]) + wb0[2, sl])

        pltpu.sync_copy(acc, shacc_all.at[sid])

    plsc.subcore_barrier()

    wblk = jnp.where(cgrp == NRG - 1, CG_OWN + 16, CG_OWN)
    alo = ((rgrp * (wblk - AW)) // (NRG - 1)) // 8 * 8
    pltpu.sync_copy(shacc_all.at[cgrp * NRG, pl.ds(alo, AW)], rd)
    gbase = cg_lo + alo

    def rbody(j, carry):
        vm, vi = carry
        v = rd[pl.ds(j * 16, 16)]
        col = gbase + j * 16 + lax.iota(jnp.int32, 16)
        upd = v > vm
        return jnp.where(upd, v, vm), jnp.where(upd, col, vi)

    vm0 = rd[pl.ds(0, 16)]
    vi0 = gbase + lax.iota(jnp.int32, 16)
    vm, vi = lax.fori_loop(1, NJA, rbody, (vm0, vi0))

    st_v[...] = vm
    st_i[...] = vi
    pltpu.sync_copy(st_v, vals_hbm.at[wid])
    pltpu.sync_copy(st_i, idx_hbm.at[wid])


_sc_top1 = pl.kernel(
    _sc_body,
    out_type=[
        jax.ShapeDtypeStruct((32, 16), jnp.float32),
        jax.ShapeDtypeStruct((32, 16), jnp.int32),
    ],
    mesh=plsc.VectorSubcoreMesh(core_axis_name="c", subcore_axis_name="s"),
    compiler_params=pltpu.CompilerParams(use_tc_tiling_on_sc=False),
    scratch_types=[
        pltpu.VMEM((D,), jnp.float32),
        pltpu.VMEM((CGW,), jnp.float32),
        pltpu.VMEM((RG, CGW), jnp.float32),
        pltpu.VMEM((RG, CGW), jnp.float32),
        pltpu.VMEM((AW,), jnp.float32),
        pltpu.VMEM((16,), jnp.float32),
        pltpu.VMEM((16,), jnp.int32),
        pltpu.VMEM_SHARED((16, CGW), jnp.float32),
        pltpu.SemaphoreType.DMA,
        pltpu.SemaphoreType.DMA,
    ],
)


def _merge_body(vals_ref, idx_ref, out_ref):
    m = jnp.max(vals_ref[...])
    out_ref[0] = jnp.min(jnp.where(vals_ref[...] == m, idx_ref[...], V))


def kernel(x, W, b):
    vals, idx = _sc_top1(x[0], W, b)
    topk_id = pl.pallas_call(
        _merge_body,
        out_specs=pl.BlockSpec(memory_space=pltpu.SMEM),
        out_shape=jax.ShapeDtypeStruct((1,), jnp.int32),
    )(vals, idx)
    return topk_id
